# baseline (speedup 1.0000x reference)
import functools

import numpy as np

import jax
import jax.numpy as jnp
from jax.experimental import pallas as pl
from jax.experimental.pallas import tpu as pltpu

_B, _C, _H, _W = 16, 768, 32, 32
_HW = _H * _W
_K = 50
_TOPK = 12
_BB = 2
_PREC = jax.lax.Precision.HIGHEST
_INTERPRET = False


_V_CACHE = {}


def _v_eps_const(seed_key):
    if seed_key not in _V_CACHE:
        try:
            with jax.ensure_compile_time_eval():
                e = np.asarray(jax.random.normal(jax.random.key(seed_key),
                                                 (_K, _B, _C, 1, 1),
                                                 jnp.float32))
            v = np.var(e.astype(np.float64), axis=0, ddof=1).reshape(_B, _C)
            _V_CACHE[seed_key] = v.astype(np.float32)
        except Exception:
            e = jax.random.normal(jax.random.key(seed_key),
                                  (_K, _B, _C, 1, 1), jnp.float32)
            return jnp.var(e, axis=0, ddof=1).reshape(_B, _C)
    return jnp.asarray(_V_CACHE[seed_key])


def _topk_indicator(score8):
    iota = jax.lax.broadcasted_iota(jnp.int32, (8, 128), 0) * 128 + \
        jax.lax.broadcasted_iota(jnp.int32, (8, 128), 1)
    s = score8
    acc = jnp.zeros((8, 128), jnp.float32)
    for _ in range(_TOPK):
        mx = jnp.max(s)
        j = jnp.min(jnp.where(s == mx, iota, jnp.int32(2 ** 30)))
        hit = iota == j
        acc = acc + hit.astype(jnp.float32)
        s = jnp.where(hit, -jnp.inf, s)
    return acc


def _wtree(w, x):
    t = (w[0:192] * x[0:192] + w[192:384] * x[192:384]
         + w[384:576] * x[384:576] + w[576:768] * x[576:768])
    t = t[0:96] + t[96:192]
    t = t[0:48] + t[48:96]
    t = t[0:24] + t[24:48]
    t = t[0:8] + t[8:16] + t[16:24]
    return jnp.sum(t, axis=0, keepdims=True)


def _pipe(feat, fp, bp, fg_t, bg_t, i, out_ref, fpp_ref, bpp_ref):
    num_fg = _wtree(fp, feat)
    num_bg = _wtree(bp, feat)
    cn = jnp.sqrt(_wtree(feat, feat))
    nf = jnp.sqrt(_wtree(fp, fp)[0, 0])
    nb = jnp.sqrt(_wtree(bp, bp)[0, 0])
    den = jnp.maximum(cn, 1e-8)
    sfg = num_fg / (den * jnp.maximum(nf, 1e-8)) * 10.0
    sbg = num_bg / (den * jnp.maximum(nb, 1e-8)) * 10.0
    out_ref[i] = jnp.concatenate([sbg, sfg], axis=0)
    m = jnp.maximum(sfg, sbg)
    efg = jnp.exp(sfg - m)
    ebg = jnp.exp(sbg - m)
    tot = efg + ebg
    fgp = efg / tot
    bgp = ebg / tot

    fm = (fgp > fg_t).astype(jnp.float32)
    bm = (bgp > bg_t).astype(jnp.float32)
    fc = jnp.sum(fm)
    bc = jnp.sum(bm)
    w2 = jnp.concatenate([fm, bm], axis=0)
    sums = jax.lax.dot_general(
        feat, w2, (((1,), (1,)), ((), ())),
        preferred_element_type=jnp.float32)
    fpp_ref[i] = sums[:, 0:1] / jnp.maximum(fc, 1.0)
    bpp_ref[i] = sums[:, 1:2] / jnp.maximum(bc, 1.0)

    @pl.when((fc == 0.0) | (bc == 0.0))
    def _fallback():
        fa = _topk_indicator(fgp.reshape(8, 128)).reshape(1, _HW)
        ba = _topk_indicator(bgp.reshape(8, 128)).reshape(1, _HW)
        w2b = jnp.concatenate([fa, ba], axis=0)
        sums_b = jax.lax.dot_general(
            feat, w2b, (((1,), (1,)), ((), ())),
            preferred_element_type=jnp.float32)
        fpp_ref[i] = jnp.where(fc > 0, sums[:, 0:1] / jnp.maximum(fc, 1.0),
                               sums_b[:, 0:1] / float(_TOPK))
        bpp_ref[i] = jnp.where(bc > 0, sums[:, 1:2] / jnp.maximum(bc, 1.0),
                               sums_b[:, 1:2] / float(_TOPK))


def _s1_body(rf_ref, df_ref, rfp_ref, rbp_ref, dfp_ref, dbp_ref, thr_ref,
             rout_ref, dout_ref, rfpp_ref, rbpp_ref, dfpp_ref, dbpp_ref):
    fg_t = thr_ref[0, 0]
    bg_t = thr_ref[0, 1]
    for i in range(_BB):
        _pipe(rf_ref[i], rfp_ref[i], rbp_ref[i], fg_t, bg_t,
              i, rout_ref, rfpp_ref, rbpp_ref)
        _pipe(df_ref[i], dfp_ref[i], dbp_ref[i], fg_t, bg_t,
              i, dout_ref, dfpp_ref, dbpp_ref)


def _s2_branch(rp, dp, wmT, wvT, v_eps, a, b):
    fused0 = a * rp + b * dp
    mean = jax.lax.dot_general(
        fused0, wmT, (((1,), (0,)), ((), ())),
        preferred_element_type=jnp.float32)
    lv = jax.lax.dot_general(
        fused0, wvT, (((1,), (0,)), ((), ())),
        preferred_element_type=jnp.float32)
    ur = jnp.exp(lv) * v_eps
    nrm = jnp.sqrt(jnp.sum(ur * ur, axis=1, keepdims=True))
    unc = 10.0 * ur / nrm
    fused = (1.0 - unc) * fused0 + rp + dp
    return fused, mean, lv


def _s2_body(rfpp_ref, rbpp_ref, dfpp_ref, dbpp_ref,
             wmf_ref, wvf_ref, wmb_ref, wvb_ref, vf_ref, vb_ref, coef_ref,
             ffp_ref, fbp_ref, mfp_ref, lfp_ref, mbp_ref, lbp_ref):
    af = coef_ref[0, 0]
    bf = coef_ref[0, 1]
    ab_ = coef_ref[0, 2]
    bb = coef_ref[0, 3]
    f, m, l = _s2_branch(rfpp_ref[...], dfpp_ref[...], wmf_ref[...],
                         wvf_ref[...], vf_ref[...], af, bf)
    ffp_ref[...] = f
    mfp_ref[...] = m
    lfp_ref[...] = l
    f, m, l = _s2_branch(rbpp_ref[...], dbpp_ref[...], wmb_ref[...],
                         wvb_ref[...], vb_ref[...], ab_, bb)
    fbp_ref[...] = f
    mbp_ref[...] = m
    lbp_ref[...] = l


def kernel(res_supp_fp, res_supp_bp, res_query_fea, dinov2_supp_fp,
           dinov2_supp_bp, dinov2_query_fea, fg_thres, bg_thres,
           alpha_fp, beta_fp, Wm_fp, Wv_fp, alpha_bp, beta_bp, Wm_bp, Wv_bp):
    f32 = jnp.float32
    rf = res_query_fea.reshape(_B, _C, _HW)
    df = dinov2_query_fea.reshape(_B, _C, _HW)
    rfp = res_supp_fp.reshape(_B, _C, 1)
    rbp = res_supp_bp.reshape(_B, _C, 1)
    dfp = dinov2_supp_fp.reshape(_B, _C, 1)
    dbp = dinov2_supp_bp.reshape(_B, _C, 1)
    thr = jnp.stack([fg_thres, bg_thres]).reshape(1, 2).astype(f32)

    feat_spec = pl.BlockSpec((_BB, _C, _HW), lambda b: (b, 0, 0))
    prot_in_spec = pl.BlockSpec((_BB, _C, 1), lambda b: (b, 0, 0))
    thr_spec = pl.BlockSpec((1, 2), lambda b: (0, 0))
    out_spec = pl.BlockSpec((_BB, 2, _HW), lambda b: (b, 0, 0))
    prot_out_spec = pl.BlockSpec((_BB, _C, 1), lambda b: (b, 0, 0))

    s1 = pl.pallas_call(
        _s1_body,
        grid=(_B // _BB,),
        in_specs=[feat_spec, feat_spec, prot_in_spec, prot_in_spec,
                  prot_in_spec, prot_in_spec, thr_spec],
        out_specs=[out_spec, out_spec, prot_out_spec, prot_out_spec,
                   prot_out_spec, prot_out_spec],
        out_shape=[
            jax.ShapeDtypeStruct((_B, 2, _HW), f32),
            jax.ShapeDtypeStruct((_B, 2, _HW), f32),
            jax.ShapeDtypeStruct((_B, _C, 1), f32),
            jax.ShapeDtypeStruct((_B, _C, 1), f32),
            jax.ShapeDtypeStruct((_B, _C, 1), f32),
            jax.ShapeDtypeStruct((_B, _C, 1), f32),
        ],
        compiler_params=pltpu.CompilerParams(
            dimension_semantics=("parallel",)),
        interpret=_INTERPRET,
    )
    rout, dout, rfpp, rbpp, dfpp, dbpp = s1(rf, df, rfp, rbp, dfp, dbp, thr)

    coef = jnp.stack([alpha_fp, beta_fp, alpha_bp, beta_bp]).reshape(1, 4)
    coef = coef.astype(f32)
    v_fp = _v_eps_const(42)
    v_bp = _v_eps_const(43)

    s2 = pl.pallas_call(
        _s2_body,
        out_shape=[jax.ShapeDtypeStruct((_B, _C), f32)] * 6,
        interpret=_INTERPRET,
    )
    ffp, fbp, mfp, lfp, mbp, lbp = s2(
        rfpp.reshape(_B, _C), rbpp.reshape(_B, _C),
        dfpp.reshape(_B, _C), dbpp.reshape(_B, _C),
        Wm_fp.T, Wv_fp.T, Wm_bp.T, Wv_bp.T, v_fp, v_bp, coef)

    q4 = lambda x: x.reshape(_B, _C, 1, 1)
    return (rout.reshape(_B, 2, _H, _W), dout.reshape(_B, 2, _H, _W),
            q4(ffp), q4(fbp), q4(mfp), q4(lfp), q4(mbp), q4(lbp))


# speedup vs baseline: 1.2296x; 1.2296x over previous
import numpy as np

import jax
import jax.numpy as jnp
from jax.experimental import pallas as pl
from jax.experimental.pallas import tpu as pltpu

_B, _C, _H, _W = 16, 768, 32, 32
_HW = _H * _W
_K = 50
_TOPK = 12
_BB = 2
_PREC = jax.lax.Precision.HIGHEST
_INTERPRET = False


_V_CACHE = {}


def _v_eps_const(seed_key):
    if seed_key not in _V_CACHE:
        try:
            with jax.ensure_compile_time_eval():
                e = np.asarray(jax.random.normal(jax.random.key(seed_key),
                                                 (_K, _B, _C, 1, 1),
                                                 jnp.float32))
            v = np.var(e.astype(np.float64), axis=0, ddof=1).reshape(_B, _C)
            _V_CACHE[seed_key] = v.astype(np.float32)
        except Exception:
            e = jax.random.normal(jax.random.key(seed_key),
                                  (_K, _B, _C, 1, 1), jnp.float32)
            return jnp.var(e, axis=0, ddof=1).reshape(_B, _C)
    return jnp.asarray(_V_CACHE[seed_key])


def _topk_indicator(score8):
    iota = jax.lax.broadcasted_iota(jnp.int32, (8, 128), 0) * 128 + \
        jax.lax.broadcasted_iota(jnp.int32, (8, 128), 1)
    s = score8
    acc = jnp.zeros((8, 128), jnp.float32)
    for _ in range(_TOPK):
        mx = jnp.max(s)
        j = jnp.min(jnp.where(s == mx, iota, jnp.int32(2 ** 30)))
        hit = iota == j
        acc = acc + hit.astype(jnp.float32)
        s = jnp.where(hit, -jnp.inf, s)
    return acc


def _pipe(X, fp, bp, fg_t, bg_t, i, out_ref, fpp_ref, bpp_ref):
    P = jnp.concatenate([fp, bp], axis=0)
    num = jax.lax.dot_general(
        X, P, (((1,), (1,)), ((), ())),
        preferred_element_type=jnp.float32, precision=_PREC)
    ones = jnp.ones((1, _C), jnp.float32)
    cn2 = jax.lax.dot_general(
        X * X, ones, (((1,), (1,)), ((), ())),
        preferred_element_type=jnp.float32, precision=_PREC)
    nf = jnp.sqrt(jnp.sum(fp * fp))
    nb = jnp.sqrt(jnp.sum(bp * bp))
    den = jnp.maximum(jnp.sqrt(cn2), 1e-8)
    sfg = num[:, 0:1] / (den * jnp.maximum(nf, 1e-8)) * 10.0
    sbg = num[:, 1:2] / (den * jnp.maximum(nb, 1e-8)) * 10.0
    out_ref[i] = jnp.concatenate([sbg, sfg], axis=1)

    sfg8 = sfg.reshape(8, 128)
    sbg8 = sbg.reshape(8, 128)
    m = jnp.maximum(sfg8, sbg8)
    efg = jnp.exp(sfg8 - m)
    ebg = jnp.exp(sbg8 - m)
    tot = efg + ebg
    fgp = efg / tot
    bgp = ebg / tot
    fm8 = (fgp > fg_t).astype(jnp.float32)
    bm8 = (bgp > bg_t).astype(jnp.float32)
    fc = jnp.sum(fm8)
    bc = jnp.sum(bm8)
    w2 = jnp.concatenate([fm8.reshape(1, _HW), bm8.reshape(1, _HW)],
                         axis=0)
    sums = jax.lax.dot_general(
        w2, X, (((1,), (0,)), ((), ())),
        preferred_element_type=jnp.float32)
    fpp_ref[i] = sums[0:1] / jnp.maximum(fc, 1.0)
    bpp_ref[i] = sums[1:2] / jnp.maximum(bc, 1.0)

    @pl.when((fc == 0.0) | (bc == 0.0))
    def _fallback():
        fa = _topk_indicator(fgp).reshape(1, _HW)
        ba = _topk_indicator(bgp).reshape(1, _HW)
        w2b = jnp.concatenate([fa, ba], axis=0)
        sums_b = jax.lax.dot_general(
            w2b, X, (((1,), (0,)), ((), ())),
            preferred_element_type=jnp.float32)
        fpp_ref[i] = jnp.where(fc > 0, sums[0:1] / jnp.maximum(fc, 1.0),
                               sums_b[0:1] / float(_TOPK))
        bpp_ref[i] = jnp.where(bc > 0, sums[1:2] / jnp.maximum(bc, 1.0),
                               sums_b[1:2] / float(_TOPK))


def _s1_body(rf_ref, df_ref, rfp_ref, rbp_ref, dfp_ref, dbp_ref, thr_ref,
             rout_ref, dout_ref, rfpp_ref, rbpp_ref, dfpp_ref, dbpp_ref):
    fg_t = thr_ref[0, 0]
    bg_t = thr_ref[0, 1]
    for i in range(_BB):
        _pipe(rf_ref[i], rfp_ref[i], rbp_ref[i], fg_t, bg_t,
              i, rout_ref, rfpp_ref, rbpp_ref)
        _pipe(df_ref[i], dfp_ref[i], dbp_ref[i], fg_t, bg_t,
              i, dout_ref, dfpp_ref, dbpp_ref)


def _s2_branch(rp, dp, wmT, wvT, v_eps, a, b):
    fused0 = a * rp + b * dp
    mean = jax.lax.dot_general(
        fused0, wmT, (((1,), (0,)), ((), ())),
        preferred_element_type=jnp.float32)
    lv = jax.lax.dot_general(
        fused0, wvT, (((1,), (0,)), ((), ())),
        preferred_element_type=jnp.float32)
    ur = jnp.exp(lv) * v_eps
    nrm = jnp.sqrt(jnp.sum(ur * ur, axis=1, keepdims=True))
    unc = 10.0 * ur / nrm
    fused = (1.0 - unc) * fused0 + rp + dp
    return fused, mean, lv


def _s2_body(rfpp_ref, rbpp_ref, dfpp_ref, dbpp_ref,
             wmf_ref, wvf_ref, wmb_ref, wvb_ref, vf_ref, vb_ref, coef_ref,
             ffp_ref, fbp_ref, mfp_ref, lfp_ref, mbp_ref, lbp_ref):
    af = coef_ref[0, 0]
    bf = coef_ref[0, 1]
    ab_ = coef_ref[0, 2]
    bb = coef_ref[0, 3]
    f, m, l = _s2_branch(rfpp_ref[...], dfpp_ref[...], wmf_ref[...],
                         wvf_ref[...], vf_ref[...], af, bf)
    ffp_ref[...] = f
    mfp_ref[...] = m
    lfp_ref[...] = l
    f, m, l = _s2_branch(rbpp_ref[...], dbpp_ref[...], wmb_ref[...],
                         wvb_ref[...], vb_ref[...], ab_, bb)
    fbp_ref[...] = f
    mbp_ref[...] = m
    lbp_ref[...] = l


def kernel(res_supp_fp, res_supp_bp, res_query_fea, dinov2_supp_fp,
           dinov2_supp_bp, dinov2_query_fea, fg_thres, bg_thres,
           alpha_fp, beta_fp, Wm_fp, Wv_fp, alpha_bp, beta_bp, Wm_bp, Wv_bp):
    f32 = jnp.float32
    rf = jnp.transpose(res_query_fea, (0, 2, 3, 1)).reshape(_B, _HW, _C)
    df = jnp.transpose(dinov2_query_fea, (0, 2, 3, 1)).reshape(_B, _HW, _C)
    rfp = res_supp_fp.reshape(_B, 1, _C)
    rbp = res_supp_bp.reshape(_B, 1, _C)
    dfp = dinov2_supp_fp.reshape(_B, 1, _C)
    dbp = dinov2_supp_bp.reshape(_B, 1, _C)
    thr = jnp.stack([fg_thres, bg_thres]).reshape(1, 2).astype(f32)

    feat_spec = pl.BlockSpec((_BB, _HW, _C), lambda b: (b, 0, 0))
    prot_in_spec = pl.BlockSpec((_BB, 1, _C), lambda b: (b, 0, 0))
    thr_spec = pl.BlockSpec((1, 2), lambda b: (0, 0))
    out_spec = pl.BlockSpec((_BB, _HW, 2), lambda b: (b, 0, 0))
    prot_out_spec = pl.BlockSpec((_BB, 1, _C), lambda b: (b, 0, 0))

    s1 = pl.pallas_call(
        _s1_body,
        grid=(_B // _BB,),
        in_specs=[feat_spec, feat_spec, prot_in_spec, prot_in_spec,
                  prot_in_spec, prot_in_spec, thr_spec],
        out_specs=[out_spec, out_spec, prot_out_spec, prot_out_spec,
                   prot_out_spec, prot_out_spec],
        out_shape=[
            jax.ShapeDtypeStruct((_B, _HW, 2), f32),
            jax.ShapeDtypeStruct((_B, _HW, 2), f32),
            jax.ShapeDtypeStruct((_B, 1, _C), f32),
            jax.ShapeDtypeStruct((_B, 1, _C), f32),
            jax.ShapeDtypeStruct((_B, 1, _C), f32),
            jax.ShapeDtypeStruct((_B, 1, _C), f32),
        ],
        compiler_params=pltpu.CompilerParams(
            dimension_semantics=("parallel",)),
        interpret=_INTERPRET,
    )
    rout, dout, rfpp, rbpp, dfpp, dbpp = s1(rf, df, rfp, rbp, dfp, dbp, thr)

    coef = jnp.stack([alpha_fp, beta_fp, alpha_bp, beta_bp]).reshape(1, 4)
    coef = coef.astype(f32)
    v_fp = _v_eps_const(42)
    v_bp = _v_eps_const(43)

    s2 = pl.pallas_call(
        _s2_body,
        out_shape=[jax.ShapeDtypeStruct((_B, _C), f32)] * 6,
        interpret=_INTERPRET,
    )
    ffp, fbp, mfp, lfp, mbp, lbp = s2(
        rfpp.reshape(_B, _C), rbpp.reshape(_B, _C),
        dfpp.reshape(_B, _C), dbpp.reshape(_B, _C),
        Wm_fp.T, Wv_fp.T, Wm_bp.T, Wv_bp.T, v_fp, v_bp, coef)

    qout = lambda x: x.transpose(0, 2, 1).reshape(_B, 2, _H, _W)
    q4 = lambda x: x.reshape(_B, _C, 1, 1)
    return (qout(rout), qout(dout),
            q4(ffp), q4(fbp), q4(mfp), q4(lfp), q4(mbp), q4(lbp))


# speedup vs baseline: 2.2047x; 1.7931x over previous
import numpy as np

import jax
import jax.numpy as jnp
from jax.experimental import pallas as pl
from jax.experimental.pallas import tpu as pltpu

_B, _C, _H, _W = 16, 768, 32, 32
_HW = _H * _W
_K = 50
_TOPK = 12
_BB = 2
_PREC = jax.lax.Precision.HIGH
_INTERPRET = False


_V_CACHE = {}


def _v_eps_const(seed_key):
    if seed_key not in _V_CACHE:
        try:
            with jax.ensure_compile_time_eval():
                e = np.asarray(jax.random.normal(jax.random.key(seed_key),
                                                 (_K, _B, _C, 1, 1),
                                                 jnp.float32))
            v = np.var(e.astype(np.float64), axis=0, ddof=1).reshape(_B, _C)
            _V_CACHE[seed_key] = v.astype(np.float32)
        except Exception:
            e = jax.random.normal(jax.random.key(seed_key),
                                  (_K, _B, _C, 1, 1), jnp.float32)
            return jnp.var(e, axis=0, ddof=1).reshape(_B, _C)
    return jnp.asarray(_V_CACHE[seed_key])


def _topk_indicator(score8):
    iota = jax.lax.broadcasted_iota(jnp.int32, (8, 128), 0) * 128 + \
        jax.lax.broadcasted_iota(jnp.int32, (8, 128), 1)
    s = score8
    acc = jnp.zeros((8, 128), jnp.float32)
    for _ in range(_TOPK):
        mx = jnp.max(s)
        j = jnp.min(jnp.where(s == mx, iota, jnp.int32(2 ** 30)))
        hit = iota == j
        acc = acc + hit.astype(jnp.float32)
        s = jnp.where(hit, -jnp.inf, s)
    return acc


def _pipe(X, fp, bp, fg_t, bg_t, i, out_ref, fpp_ref, bpp_ref):
    def lane_tree(w):
        t = w[:, 0:384] * X[:, 0:384] + w[:, 384:768] * X[:, 384:768]
        t = t[:, 0:128] + t[:, 128:256] + t[:, 256:384]
        return jnp.sum(t, axis=1, keepdims=True)

    num_fg = lane_tree(fp)
    num_bg = lane_tree(bp)
    cn2 = lane_tree(X)
    nf = jnp.sqrt(jnp.sum(fp * fp))
    nb = jnp.sqrt(jnp.sum(bp * bp))
    den = jnp.maximum(jnp.sqrt(cn2), 1e-8)
    sfg = num_fg / (den * jnp.maximum(nf, 1e-8)) * 10.0
    sbg = num_bg / (den * jnp.maximum(nb, 1e-8)) * 10.0
    out_ref[i] = jnp.concatenate([sbg, sfg], axis=1)

    sfg8 = sfg.reshape(8, 128)
    sbg8 = sbg.reshape(8, 128)
    m = jnp.maximum(sfg8, sbg8)
    efg = jnp.exp(sfg8 - m)
    ebg = jnp.exp(sbg8 - m)
    tot = efg + ebg
    fgp = efg / tot
    bgp = ebg / tot
    fm8 = (fgp > fg_t).astype(jnp.float32)
    bm8 = (bgp > bg_t).astype(jnp.float32)
    fc = jnp.sum(fm8)
    bc = jnp.sum(bm8)
    w2 = jnp.concatenate([fm8.reshape(1, _HW), bm8.reshape(1, _HW)],
                         axis=0)
    sums = jax.lax.dot_general(
        w2, X, (((1,), (0,)), ((), ())),
        preferred_element_type=jnp.float32)
    fpp_ref[i] = sums[0:1] / jnp.maximum(fc, 1.0)
    bpp_ref[i] = sums[1:2] / jnp.maximum(bc, 1.0)

    @pl.when((fc == 0.0) | (bc == 0.0))
    def _fallback():
        fa = _topk_indicator(fgp).reshape(1, _HW)
        ba = _topk_indicator(bgp).reshape(1, _HW)
        w2b = jnp.concatenate([fa, ba], axis=0)
        sums_b = jax.lax.dot_general(
            w2b, X, (((1,), (0,)), ((), ())),
            preferred_element_type=jnp.float32)
        fpp_ref[i] = jnp.where(fc > 0, sums[0:1] / jnp.maximum(fc, 1.0),
                               sums_b[0:1] / float(_TOPK))
        bpp_ref[i] = jnp.where(bc > 0, sums[1:2] / jnp.maximum(bc, 1.0),
                               sums_b[1:2] / float(_TOPK))


def _s1_body(rf_ref, df_ref, rfp_ref, rbp_ref, dfp_ref, dbp_ref, thr_ref,
             rout_ref, dout_ref, rfpp_ref, rbpp_ref, dfpp_ref, dbpp_ref):
    fg_t = thr_ref[0, 0]
    bg_t = thr_ref[0, 1]
    for i in range(_BB):
        _pipe(rf_ref[i], rfp_ref[i], rbp_ref[i], fg_t, bg_t,
              i, rout_ref, rfpp_ref, rbpp_ref)
        _pipe(df_ref[i], dfp_ref[i], dbp_ref[i], fg_t, bg_t,
              i, dout_ref, dfpp_ref, dbpp_ref)


def _s2_branch(rp, dp, wmT, wvT, v_eps, a, b):
    fused0 = a * rp + b * dp
    mean = jax.lax.dot_general(
        fused0, wmT, (((1,), (0,)), ((), ())),
        preferred_element_type=jnp.float32)
    lv = jax.lax.dot_general(
        fused0, wvT, (((1,), (0,)), ((), ())),
        preferred_element_type=jnp.float32)
    ur = jnp.exp(lv) * v_eps
    nrm = jnp.sqrt(jnp.sum(ur * ur, axis=1, keepdims=True))
    unc = 10.0 * ur / nrm
    fused = (1.0 - unc) * fused0 + rp + dp
    return fused, mean, lv


def _s2_body(rfpp_ref, rbpp_ref, dfpp_ref, dbpp_ref,
             wmf_ref, wvf_ref, wmb_ref, wvb_ref, vf_ref, vb_ref, coef_ref,
             ffp_ref, fbp_ref, mfp_ref, lfp_ref, mbp_ref, lbp_ref):
    af = coef_ref[0, 0]
    bf = coef_ref[0, 1]
    ab_ = coef_ref[0, 2]
    bb = coef_ref[0, 3]
    f, m, l = _s2_branch(rfpp_ref[...], dfpp_ref[...], wmf_ref[...],
                         wvf_ref[...], vf_ref[...], af, bf)
    ffp_ref[...] = f
    mfp_ref[...] = m
    lfp_ref[...] = l
    f, m, l = _s2_branch(rbpp_ref[...], dbpp_ref[...], wmb_ref[...],
                         wvb_ref[...], vb_ref[...], ab_, bb)
    fbp_ref[...] = f
    mbp_ref[...] = m
    lbp_ref[...] = l


def kernel(res_supp_fp, res_supp_bp, res_query_fea, dinov2_supp_fp,
           dinov2_supp_bp, dinov2_query_fea, fg_thres, bg_thres,
           alpha_fp, beta_fp, Wm_fp, Wv_fp, alpha_bp, beta_bp, Wm_bp, Wv_bp):
    f32 = jnp.float32
    rf = jnp.transpose(res_query_fea, (0, 2, 3, 1)).reshape(_B, _HW, _C)
    df = jnp.transpose(dinov2_query_fea, (0, 2, 3, 1)).reshape(_B, _HW, _C)
    rfp = res_supp_fp.reshape(_B, 1, _C)
    rbp = res_supp_bp.reshape(_B, 1, _C)
    dfp = dinov2_supp_fp.reshape(_B, 1, _C)
    dbp = dinov2_supp_bp.reshape(_B, 1, _C)
    thr = jnp.stack([fg_thres, bg_thres]).reshape(1, 2).astype(f32)

    feat_spec = pl.BlockSpec((_BB, _HW, _C), lambda b: (b, 0, 0))
    prot_in_spec = pl.BlockSpec((_BB, 1, _C), lambda b: (b, 0, 0))
    thr_spec = pl.BlockSpec((1, 2), lambda b: (0, 0))
    out_spec = pl.BlockSpec((_BB, _HW, 2), lambda b: (b, 0, 0))
    prot_out_spec = pl.BlockSpec((_BB, 1, _C), lambda b: (b, 0, 0))

    s1 = pl.pallas_call(
        _s1_body,
        grid=(_B // _BB,),
        in_specs=[feat_spec, feat_spec, prot_in_spec, prot_in_spec,
                  prot_in_spec, prot_in_spec, thr_spec],
        out_specs=[out_spec, out_spec, prot_out_spec, prot_out_spec,
                   prot_out_spec, prot_out_spec],
        out_shape=[
            jax.ShapeDtypeStruct((_B, _HW, 2), f32),
            jax.ShapeDtypeStruct((_B, _HW, 2), f32),
            jax.ShapeDtypeStruct((_B, 1, _C), f32),
            jax.ShapeDtypeStruct((_B, 1, _C), f32),
            jax.ShapeDtypeStruct((_B, 1, _C), f32),
            jax.ShapeDtypeStruct((_B, 1, _C), f32),
        ],
        compiler_params=pltpu.CompilerParams(
            dimension_semantics=("parallel",)),
        interpret=_INTERPRET,
    )
    rout, dout, rfpp, rbpp, dfpp, dbpp = s1(rf, df, rfp, rbp, dfp, dbp, thr)

    coef = jnp.stack([alpha_fp, beta_fp, alpha_bp, beta_bp]).reshape(1, 4)
    coef = coef.astype(f32)
    v_fp = _v_eps_const(42)
    v_bp = _v_eps_const(43)

    s2 = pl.pallas_call(
        _s2_body,
        out_shape=[jax.ShapeDtypeStruct((_B, _C), f32)] * 6,
        interpret=_INTERPRET,
    )
    ffp, fbp, mfp, lfp, mbp, lbp = s2(
        rfpp.reshape(_B, _C), rbpp.reshape(_B, _C),
        dfpp.reshape(_B, _C), dbpp.reshape(_B, _C),
        Wm_fp.T, Wv_fp.T, Wm_bp.T, Wv_bp.T, v_fp, v_bp, coef)

    qout = lambda x: x.transpose(0, 2, 1).reshape(_B, 2, _H, _W)
    q4 = lambda x: x.reshape(_B, _C, 1, 1)
    return (qout(rout), qout(dout),
            q4(ffp), q4(fbp), q4(mfp), q4(lfp), q4(mbp), q4(lbp))


# speedup vs baseline: 2.4000x; 1.0886x over previous
import numpy as np

import jax
import jax.numpy as jnp
from jax.experimental import pallas as pl
from jax.experimental.pallas import tpu as pltpu

_B, _C, _H, _W = 16, 768, 32, 32
_HW = _H * _W
_K = 50
_TOPK = 12
_BB = 2
_PREC = jax.lax.Precision.HIGH
_INTERPRET = False


_V_CACHE = {}


def _v_eps_const(seed_key):
    if seed_key not in _V_CACHE:
        try:
            with jax.ensure_compile_time_eval():
                e = np.asarray(jax.random.normal(jax.random.key(seed_key),
                                                 (_K, _B, _C, 1, 1),
                                                 jnp.float32))
            v = np.var(e.astype(np.float64), axis=0, ddof=1).reshape(_B, _C)
            _V_CACHE[seed_key] = v.astype(np.float32)
        except Exception:
            e = jax.random.normal(jax.random.key(seed_key),
                                  (_K, _B, _C, 1, 1), jnp.float32)
            return jnp.var(e, axis=0, ddof=1).reshape(_B, _C)
    return jnp.asarray(_V_CACHE[seed_key])


def _topk_indicator(score8):
    iota = jax.lax.broadcasted_iota(jnp.int32, (8, 128), 0) * 128 + \
        jax.lax.broadcasted_iota(jnp.int32, (8, 128), 1)
    s = score8
    acc = jnp.zeros((8, 128), jnp.float32)
    for _ in range(_TOPK):
        mx = jnp.max(s)
        j = jnp.min(jnp.where(s == mx, iota, jnp.int32(2 ** 30)))
        hit = iota == j
        acc = acc + hit.astype(jnp.float32)
        s = jnp.where(hit, -jnp.inf, s)
    return acc


def _pipe(X, fp, bp, fg_t, bg_t, i, out_ref, fpp_ref, bpp_ref):
    def lane_tree(w):
        t = w[:, 0:384] * X[:, 0:384] + w[:, 384:768] * X[:, 384:768]
        t = t[:, 0:128] + t[:, 128:256] + t[:, 256:384]
        return jnp.sum(t, axis=1, keepdims=True)

    num_fg = lane_tree(fp)
    num_bg = lane_tree(bp)
    cn2 = lane_tree(X)
    nf = jnp.sqrt(jnp.sum(fp * fp))
    nb = jnp.sqrt(jnp.sum(bp * bp))
    den = jnp.maximum(jnp.sqrt(cn2), 1e-8)
    sfg = num_fg / (den * jnp.maximum(nf, 1e-8)) * 10.0
    sbg = num_bg / (den * jnp.maximum(nb, 1e-8)) * 10.0
    out_ref[i] = jnp.concatenate([sbg, sfg], axis=1)

    sfg8 = sfg.reshape(8, 128)
    sbg8 = sbg.reshape(8, 128)
    m = jnp.maximum(sfg8, sbg8)
    efg = jnp.exp(sfg8 - m)
    ebg = jnp.exp(sbg8 - m)
    tot = efg + ebg
    fgp = efg / tot
    bgp = ebg / tot
    fm8 = (fgp > fg_t).astype(jnp.float32)
    bm8 = (bgp > bg_t).astype(jnp.float32)
    fc = jnp.sum(fm8)
    bc = jnp.sum(bm8)
    w2 = jnp.concatenate([fm8.reshape(1, _HW), bm8.reshape(1, _HW)],
                         axis=0)
    sums = jax.lax.dot_general(
        w2, X, (((1,), (0,)), ((), ())),
        preferred_element_type=jnp.float32)
    fpp_ref[i] = sums[0:1] / jnp.maximum(fc, 1.0)
    bpp_ref[i] = sums[1:2] / jnp.maximum(bc, 1.0)

    @pl.when((fc == 0.0) | (bc == 0.0))
    def _fallback():
        fa = _topk_indicator(fgp).reshape(1, _HW)
        ba = _topk_indicator(bgp).reshape(1, _HW)
        w2b = jnp.concatenate([fa, ba], axis=0)
        sums_b = jax.lax.dot_general(
            w2b, X, (((1,), (0,)), ((), ())),
            preferred_element_type=jnp.float32)
        fpp_ref[i] = jnp.where(fc > 0, sums[0:1] / jnp.maximum(fc, 1.0),
                               sums_b[0:1] / float(_TOPK))
        bpp_ref[i] = jnp.where(bc > 0, sums[1:2] / jnp.maximum(bc, 1.0),
                               sums_b[1:2] / float(_TOPK))


def _s1_body(rf_ref, df_ref, rfp_ref, rbp_ref, dfp_ref, dbp_ref, thr_ref,
             rout_ref, dout_ref, rfpp_ref, rbpp_ref, dfpp_ref, dbpp_ref):
    fg_t = thr_ref[0, 0]
    bg_t = thr_ref[0, 1]
    for i in range(_BB):
        _pipe(rf_ref[i], rfp_ref[i], rbp_ref[i], fg_t, bg_t,
              i, rout_ref, rfpp_ref, rbpp_ref)
        _pipe(df_ref[i], dfp_ref[i], dbp_ref[i], fg_t, bg_t,
              i, dout_ref, dfpp_ref, dbpp_ref)


def _s2_branch(rp, dp, wmT, wvT, v_eps, a, b):
    fused0 = a * rp + b * dp
    mean = jax.lax.dot_general(
        fused0, wmT, (((1,), (1,)), ((), ())),
        preferred_element_type=jnp.float32)
    lv = jax.lax.dot_general(
        fused0, wvT, (((1,), (1,)), ((), ())),
        preferred_element_type=jnp.float32)
    ur = jnp.exp(lv) * v_eps
    nrm = jnp.sqrt(jnp.sum(ur * ur, axis=1, keepdims=True))
    unc = 10.0 * ur / nrm
    fused = (1.0 - unc) * fused0 + rp + dp
    return fused, mean, lv


def _s2_body(rfpp_ref, rbpp_ref, dfpp_ref, dbpp_ref,
             wmf_ref, wvf_ref, wmb_ref, wvb_ref, vf_ref, vb_ref, coef_ref,
             ffp_ref, fbp_ref, mfp_ref, lfp_ref, mbp_ref, lbp_ref):
    af = coef_ref[0, 0]
    bf = coef_ref[0, 1]
    ab_ = coef_ref[0, 2]
    bb = coef_ref[0, 3]
    f, m, l = _s2_branch(rfpp_ref[...], dfpp_ref[...], wmf_ref[...],
                         wvf_ref[...], vf_ref[...], af, bf)
    ffp_ref[...] = f
    mfp_ref[...] = m
    lfp_ref[...] = l
    f, m, l = _s2_branch(rbpp_ref[...], dbpp_ref[...], wmb_ref[...],
                         wvb_ref[...], vb_ref[...], ab_, bb)
    fbp_ref[...] = f
    mbp_ref[...] = m
    lbp_ref[...] = l


def kernel(res_supp_fp, res_supp_bp, res_query_fea, dinov2_supp_fp,
           dinov2_supp_bp, dinov2_query_fea, fg_thres, bg_thres,
           alpha_fp, beta_fp, Wm_fp, Wv_fp, alpha_bp, beta_bp, Wm_bp, Wv_bp):
    f32 = jnp.float32
    rf = jnp.transpose(res_query_fea, (0, 2, 3, 1)).reshape(_B, _HW, _C)
    df = jnp.transpose(dinov2_query_fea, (0, 2, 3, 1)).reshape(_B, _HW, _C)
    rfp = res_supp_fp.reshape(_B, 1, _C)
    rbp = res_supp_bp.reshape(_B, 1, _C)
    dfp = dinov2_supp_fp.reshape(_B, 1, _C)
    dbp = dinov2_supp_bp.reshape(_B, 1, _C)
    thr = jnp.stack([fg_thres, bg_thres]).reshape(1, 2).astype(f32)

    feat_spec = pl.BlockSpec((_BB, _HW, _C), lambda b: (b, 0, 0))
    prot_in_spec = pl.BlockSpec((_BB, 1, _C), lambda b: (b, 0, 0))
    thr_spec = pl.BlockSpec((1, 2), lambda b: (0, 0))
    out_spec = pl.BlockSpec((_BB, _HW, 2), lambda b: (b, 0, 0))
    prot_out_spec = pl.BlockSpec((_BB, 1, _C), lambda b: (b, 0, 0))

    s1 = pl.pallas_call(
        _s1_body,
        grid=(_B // _BB,),
        in_specs=[feat_spec, feat_spec, prot_in_spec, prot_in_spec,
                  prot_in_spec, prot_in_spec, thr_spec],
        out_specs=[out_spec, out_spec, prot_out_spec, prot_out_spec,
                   prot_out_spec, prot_out_spec],
        out_shape=[
            jax.ShapeDtypeStruct((_B, _HW, 2), f32),
            jax.ShapeDtypeStruct((_B, _HW, 2), f32),
            jax.ShapeDtypeStruct((_B, 1, _C), f32),
            jax.ShapeDtypeStruct((_B, 1, _C), f32),
            jax.ShapeDtypeStruct((_B, 1, _C), f32),
            jax.ShapeDtypeStruct((_B, 1, _C), f32),
        ],
        compiler_params=pltpu.CompilerParams(
            dimension_semantics=("parallel",)),
        interpret=_INTERPRET,
    )
    rout, dout, rfpp, rbpp, dfpp, dbpp = s1(rf, df, rfp, rbp, dfp, dbp, thr)

    coef = jnp.stack([alpha_fp, beta_fp, alpha_bp, beta_bp]).reshape(1, 4)
    coef = coef.astype(f32)
    v_fp = _v_eps_const(42)
    v_bp = _v_eps_const(43)

    s2 = pl.pallas_call(
        _s2_body,
        out_shape=[jax.ShapeDtypeStruct((_B, _C), f32)] * 6,
        interpret=_INTERPRET,
    )
    ffp, fbp, mfp, lfp, mbp, lbp = s2(
        rfpp.reshape(_B, _C), rbpp.reshape(_B, _C),
        dfpp.reshape(_B, _C), dbpp.reshape(_B, _C),
        Wm_fp, Wv_fp, Wm_bp, Wv_bp, v_fp, v_bp, coef)

    qout = lambda x: x.transpose(0, 2, 1).reshape(_B, 2, _H, _W)
    q4 = lambda x: x.reshape(_B, _C, 1, 1)
    return (qout(rout), qout(dout),
            q4(ffp), q4(fbp), q4(mfp), q4(lfp), q4(mbp), q4(lbp))
